# baseline (speedup 1.0000x reference)
import jax
import jax.numpy as jnp
from jax.experimental import pallas as pl
from jax.experimental.pallas import tpu as pltpu

_N = 16384
_D = 64
_BS = 2048
_BI = 128
_NBUF = 4


def _support_body(x_ref, w_ref, s_ref):
    s_ref[...] = jnp.dot(x_ref[...], w_ref[...],
                         preferred_element_type=jnp.float32)


def _gcn_body(adj_hbm, s_ref, b_ref, o_ref, bufs, sems):
    i = pl.program_id(0)
    nsteps = pl.num_programs(0)

    def _copy(slot, band):
        pltpu.make_async_copy(
            adj_hbm.at[pl.ds(band * _BI, _BI), :],
            bufs.at[slot],
            sems.at[slot],
        ).start()

    @pl.when(i == 0)
    def _():
        for k in range(_NBUF - 1):
            _copy(k, k)

    nxt = i + _NBUF - 1

    @pl.when(nxt < nsteps)
    def _():
        _copy(jax.lax.rem(nxt, _NBUF), nxt)

    slot = jax.lax.rem(i, _NBUF)
    pltpu.make_async_copy(
        adj_hbm.at[pl.ds(i * _BI, _BI), :],
        bufs.at[slot],
        sems.at[slot],
    ).wait()
    o_ref[...] = (jnp.dot(bufs[slot], s_ref[...],
                          preferred_element_type=jnp.float32)
                  + b_ref[...])


def kernel(input_features, adj, weight, bias):
    support = jnp.dot(input_features, weight)

    out = pl.pallas_call(
        _gcn_body,
        grid=(_N // _BI,),
        in_specs=[
            pl.BlockSpec(memory_space=pltpu.MemorySpace.HBM),
            pl.BlockSpec((_N, _D), lambda i: (0, 0)),
            pl.BlockSpec((1, _D), lambda i: (0, 0)),
        ],
        out_specs=pl.BlockSpec((_BI, _D), lambda i: (i, 0)),
        out_shape=jax.ShapeDtypeStruct((_N, _D), jnp.float32),
        scratch_shapes=[
            pltpu.VMEM((_NBUF, _BI, _N), jnp.float32),
            pltpu.SemaphoreType.DMA((_NBUF,)),
        ],
        compiler_params=pltpu.CompilerParams(
            dimension_semantics=("arbitrary",)),
    )(adj, support, bias.reshape(1, _D))
    return out
